# baseline (speedup 1.0000x reference)
import functools

import jax
import jax.numpy as jnp
from jax import lax
from jax.experimental import pallas as pl
from jax.experimental.pallas import tpu as pltpu
from jax.experimental.pallas import tpu_sc as plsc

F = 26
V = 100000
D = 32
B = 16384
NC, NS, L = 2, 16, 16
NW = NC * NS
ZR = F * V // 4
RC = 768
KPF = V // RC
NTASK = F * KPF
BPW = B // NW
BC = 128

_params = pltpu.CompilerParams(use_tc_tiling_on_sc=True,
                               needs_layout_passes=False)


def _relayout(src, dst, nrow4):
    def row4(r4, c2):
        for sub in range(4):
            rr = r4 * 4 + sub
            for c0 in (0, L):
                vals = plsc.load_gather(
                    src, [c0 + lax.iota(jnp.int32, L),
                          jnp.full((L,), rr, jnp.int32)])
                dst[r4, pl.ds(sub * 32 + c0, L)] = vals
        return c2
    lax.fori_loop(0, nrow4, row4, 0)


def _body_a(tabT, z, src, dst, tailbuf, sem):
    wid = lax.axis_index("s") * NC + lax.axis_index("c")

    def task(i, carry):
        t = wid + NW * i

        @pl.when(t < NTASK)
        def _():
            f = t // KPF
            k = lax.rem(t, KPF)
            pltpu.sync_copy(tabT.at[f, :, pl.ds(k * RC, RC)], src)
            _relayout(src, dst, RC // 4)
            zr0 = f * (V // 4) + k * (RC // 4)
            pltpu.sync_copy(dst, z.at[pl.ds(zr0, RC // 4)])
        return carry

    lax.fori_loop(0, (NTASK + NW - 1) // NW, task, 0)

    @pl.when(wid < F)
    def _():
        f = wid
        pltpu.sync_copy(tabT.at[f, :, pl.ds(V - 160, 128)], src.at[:, pl.ds(0, 128)])
        pltpu.sync_copy(tabT.at[f, :, pl.ds(V - 32, 32)], tailbuf)
        _relayout(src, dst, 32)

        def row4t(r4, c2):
            for sub in range(4):
                rr = r4 * 4 + sub
                for c0 in (0, L):
                    vals = plsc.load_gather(
                        tailbuf, [c0 + lax.iota(jnp.int32, L),
                                  jnp.full((L,), rr, jnp.int32)])
                    dst[32 + r4, pl.ds(sub * 32 + c0, L)] = vals
            return c2
        lax.fori_loop(0, 8, row4t, 0)
        pltpu.sync_copy(dst.at[pl.ds(0, 40)],
                        z.at[pl.ds(f * (V // 4) + (V - 160) // 4, 40)])


def _stage_idx(xbufT, zidx, rem, f):
    for g in range(BC // L):
        xv = xbufT[f, pl.ds(g * L, L)]
        idx = xv + f * V
        zidx[pl.ds(g * L, L)] = lax.shift_right_logical(idx, 2)
        rem[pl.ds(g * L, L)] = lax.bitwise_and(idx, 3)


def _extract(zbuf, rem, obuf):
    for g in range(BC // L):
        iv = g * L + lax.iota(jnp.int32, L)
        zcol = rem[pl.ds(g * L, L)] * 32
        for c in range(32):
            vals = plsc.load_gather(zbuf, [iv, zcol + c])
            obuf[c, pl.ds(g * L, L)] = vals


def _body_b(xT, z_hbm, outT, xbufT, zb0, zb1, zi0, zi1, rm0, rm1,
            ob0, ob1, gs0, gs1, os0, os1):
    wid = lax.axis_index("s") * NC + lax.axis_index("c")
    b0w = wid * BPW

    def gather(zi, zb, sem):
        return pltpu.make_async_copy(z_hbm.at[zi], zb, sem)

    def out_copy(ob, f, b0, sem):
        return pltpu.make_async_copy(
            ob, outT.at[pl.ds(f * 32, 32), pl.ds(b0, BC)], sem)

    def bchunk(bc, carry):
        b0 = b0w + bc * BC
        pltpu.sync_copy(xT.at[:, pl.ds(b0, BC)], xbufT)

        _stage_idx(xbufT, zi0, rm0, 0)
        gather(zi0, zb0, gs0).start()

        def fpair(fp, c2):
            for q in (0, 1):
                f = fp * 2 + q
                zi, zb, rm, ob = (zi0, zb0, rm0, ob0) if q == 0 else (zi1, zb1, rm1, ob1)
                nzi, nzb, nrm = (zi1, zb1, rm1) if q == 0 else (zi0, zb0, rm0)
                ngs = gs1 if q == 0 else gs0
                osem = os0 if q == 0 else os1

                @pl.when(f + 1 < F)
                def _():
                    _stage_idx(xbufT, nzi, nrm, f + 1)
                    gather(nzi, nzb, ngs).start()
                gather(zi, zb, gs0 if q == 0 else gs1).wait()

                @pl.when(f >= 2)
                def _():
                    out_copy(ob, f - 2, b0, osem).wait()
                _extract(zb, rm, ob)
                out_copy(ob, f, b0, osem).start()
            return c2

        lax.fori_loop(0, F // 2, fpair, 0)
        out_copy(ob0, F - 2, b0, os0).wait()
        out_copy(ob1, F - 1, b0, os1).wait()
        return carry

    lax.fori_loop(0, BPW // BC, bchunk, 0)


@jax.jit
def _multi_embedding(xT, tabT):
    mesh = plsc.VectorSubcoreMesh(core_axis_name="c", subcore_axis_name="s")
    z = pl.kernel(
        _body_a, mesh=mesh, compiler_params=_params,
        out_type=jax.ShapeDtypeStruct((ZR, 128), jnp.float32),
        scratch_types=[
            pltpu.VMEM((32, RC), jnp.float32),
            pltpu.VMEM((RC // 4, 128), jnp.float32),
            pltpu.VMEM((32, 32), jnp.float32),
            pltpu.SemaphoreType.DMA,
        ],
    )(tabT)
    outT = pl.kernel(
        _body_b, mesh=mesh, compiler_params=_params,
        out_type=jax.ShapeDtypeStruct((F * D, B), jnp.float32),
        scratch_types=[
            pltpu.VMEM((F, BC), jnp.int32),
            pltpu.VMEM((BC, 128), jnp.float32),
            pltpu.VMEM((BC, 128), jnp.float32),
            pltpu.VMEM((BC,), jnp.int32),
            pltpu.VMEM((BC,), jnp.int32),
            pltpu.VMEM((BC,), jnp.int32),
            pltpu.VMEM((BC,), jnp.int32),
            pltpu.VMEM((32, BC), jnp.float32),
            pltpu.VMEM((32, BC), jnp.float32),
            pltpu.SemaphoreType.DMA,
            pltpu.SemaphoreType.DMA,
            pltpu.SemaphoreType.DMA,
            pltpu.SemaphoreType.DMA,
        ],
    )(xT, z)
    return outT


def kernel(x, tables):
    tabT = jnp.transpose(tables, (0, 2, 1))
    xT = jnp.transpose(x, (1, 0))
    outT = _multi_embedding(xT, tabT)
    return jnp.transpose(outT, (1, 0))


# speedup vs baseline: 1.1235x; 1.1235x over previous
import functools

import jax
import jax.numpy as jnp
from jax import lax
from jax.experimental import pallas as pl
from jax.experimental.pallas import tpu as pltpu
from jax.experimental.pallas import tpu_sc as plsc

F = 26
V = 100000
D = 32
B = 16384
NC, NS, L = 2, 16, 16
NW = NC * NS
ZR = F * V // 4
RC = 768
KPF = V // RC
NTASK = F * KPF
BPW = B // NW
BC = 128

_params = pltpu.CompilerParams(use_tc_tiling_on_sc=True,
                               needs_layout_passes=False)


def _relayout(src, dst, nrow4):
    def row4(r4, c2):
        for sub in range(4):
            rr = r4 * 4 + sub
            for c0 in (0, L):
                vals = plsc.load_gather(
                    src, [c0 + lax.iota(jnp.int32, L),
                          jnp.full((L,), rr, jnp.int32)])
                dst[r4, pl.ds(sub * 32 + c0, L)] = vals
        return c2
    lax.fori_loop(0, nrow4, row4, 0)


def _body_a(tabT, z, s0, s1, d0, d1, tailbuf, is0, is1, os0, os1):
    wid = lax.axis_index("s") * NC + lax.axis_index("c")
    src, dst = s0, d0

    def fk(t):
        return t // KPF, lax.rem(t, KPF)

    def in_copy(t, sb, sem):
        f, k = fk(t)
        return pltpu.make_async_copy(tabT.at[f, :, pl.ds(k * RC, RC)], sb, sem)

    def out_copy(t, db, sem):
        f, k = fk(t)
        zr0 = f * (V // 4) + k * (RC // 4)
        return pltpu.make_async_copy(db, z.at[pl.ds(zr0, RC // 4)], sem)

    NIT = (NTASK + NW - 1) // NW

    @pl.when(wid < NTASK)
    def _():
        in_copy(wid, s0, is0).start()

    def pair(i2, carry):
        for q in (0, 1):
            ii = i2 * 2 + q
            t = wid + NW * ii
            sb, db, isem, osem = (s0, d0, is0, os0) if q == 0 else (s1, d1, is1, os1)
            nsb, nisem = (s1, is1) if q == 0 else (s0, is0)

            tp = t - 2 * NW

            @pl.when(jnp.logical_and(tp >= 0, tp < NTASK))
            def _():
                out_copy(tp, db, osem).wait()

            @pl.when(t < NTASK)
            def _():
                tn = t + NW

                @pl.when(tn < NTASK)
                def _():
                    in_copy(tn, nsb, nisem).start()
                in_copy(t, sb, isem).wait()
                _relayout(sb, db, RC // 4)
                out_copy(t, db, osem).start()
        return carry

    lax.fori_loop(0, NIT // 2, pair, 0)

    for q, osem, db in ((0, os0, d0), (1, os1, d1)):
        last = wid + NW * (NIT - 2 + q)

        @pl.when(last < NTASK)
        def _():
            out_copy(last, db, osem).wait()

    @pl.when(wid < F)
    def _():
        f = wid
        pltpu.sync_copy(tabT.at[f, :, pl.ds(V - 160, 128)], src.at[:, pl.ds(0, 128)])
        pltpu.sync_copy(tabT.at[f, :, pl.ds(V - 32, 32)], tailbuf)
        _relayout(src, dst, 32)

        def row4t(r4, c2):
            for sub in range(4):
                rr = r4 * 4 + sub
                for c0 in (0, L):
                    vals = plsc.load_gather(
                        tailbuf, [c0 + lax.iota(jnp.int32, L),
                                  jnp.full((L,), rr, jnp.int32)])
                    dst[32 + r4, pl.ds(sub * 32 + c0, L)] = vals
            return c2
        lax.fori_loop(0, 8, row4t, 0)
        pltpu.sync_copy(dst.at[pl.ds(0, 40)],
                        z.at[pl.ds(f * (V // 4) + (V - 160) // 4, 40)])


def _stage_idx(xbufT, zidx, rem, f):
    for g in range(BC // L):
        xv = xbufT[f, pl.ds(g * L, L)]
        idx = xv + f * V
        zidx[pl.ds(g * L, L)] = lax.shift_right_logical(idx, 2)
        rem[pl.ds(g * L, L)] = lax.bitwise_and(idx, 3)


def _extract(zbuf, rem, obuf):
    for g in range(BC // L):
        iv = g * L + lax.iota(jnp.int32, L)
        zcol = rem[pl.ds(g * L, L)] * 32
        for c in range(32):
            vals = plsc.load_gather(zbuf, [iv, zcol + c])
            obuf[c, pl.ds(g * L, L)] = vals


def _body_b(xT, z_hbm, outT, xbufT, zb0, zb1, zi0, zi1, rm0, rm1,
            ob0, ob1, gs0, gs1, os0, os1):
    wid = lax.axis_index("s") * NC + lax.axis_index("c")
    b0w = wid * BPW

    def gather(zi, zb, sem):
        return pltpu.make_async_copy(z_hbm.at[zi], zb, sem)

    def out_copy(ob, f, b0, sem):
        return pltpu.make_async_copy(
            ob, outT.at[pl.ds(f * 32, 32), pl.ds(b0, BC)], sem)

    def bchunk(bc, carry):
        b0 = b0w + bc * BC
        pltpu.sync_copy(xT.at[:, pl.ds(b0, BC)], xbufT)

        _stage_idx(xbufT, zi0, rm0, 0)
        gather(zi0, zb0, gs0).start()

        def fpair(fp, c2):
            for q in (0, 1):
                f = fp * 2 + q
                zi, zb, rm, ob = (zi0, zb0, rm0, ob0) if q == 0 else (zi1, zb1, rm1, ob1)
                nzi, nzb, nrm = (zi1, zb1, rm1) if q == 0 else (zi0, zb0, rm0)
                ngs = gs1 if q == 0 else gs0
                osem = os0 if q == 0 else os1

                @pl.when(f + 1 < F)
                def _():
                    _stage_idx(xbufT, nzi, nrm, f + 1)
                    gather(nzi, nzb, ngs).start()
                gather(zi, zb, gs0 if q == 0 else gs1).wait()

                @pl.when(f >= 2)
                def _():
                    out_copy(ob, f - 2, b0, osem).wait()
                _extract(zb, rm, ob)
                out_copy(ob, f, b0, osem).start()
            return c2

        lax.fori_loop(0, F // 2, fpair, 0)
        out_copy(ob0, F - 2, b0, os0).wait()
        out_copy(ob1, F - 1, b0, os1).wait()
        return carry

    lax.fori_loop(0, BPW // BC, bchunk, 0)


@jax.jit
def _multi_embedding(xT, tabT):
    mesh = plsc.VectorSubcoreMesh(core_axis_name="c", subcore_axis_name="s")
    z = pl.kernel(
        _body_a, mesh=mesh, compiler_params=_params,
        out_type=jax.ShapeDtypeStruct((ZR, 128), jnp.float32),
        scratch_types=[
            pltpu.VMEM((32, RC), jnp.float32),
            pltpu.VMEM((32, RC), jnp.float32),
            pltpu.VMEM((RC // 4, 128), jnp.float32),
            pltpu.VMEM((RC // 4, 128), jnp.float32),
            pltpu.VMEM((32, 32), jnp.float32),
            pltpu.SemaphoreType.DMA,
            pltpu.SemaphoreType.DMA,
            pltpu.SemaphoreType.DMA,
            pltpu.SemaphoreType.DMA,
        ],
    )(tabT)
    outT = pl.kernel(
        _body_b, mesh=mesh, compiler_params=_params,
        out_type=jax.ShapeDtypeStruct((F * D, B), jnp.float32),
        scratch_types=[
            pltpu.VMEM((F, BC), jnp.int32),
            pltpu.VMEM((BC, 128), jnp.float32),
            pltpu.VMEM((BC, 128), jnp.float32),
            pltpu.VMEM((BC,), jnp.int32),
            pltpu.VMEM((BC,), jnp.int32),
            pltpu.VMEM((BC,), jnp.int32),
            pltpu.VMEM((BC,), jnp.int32),
            pltpu.VMEM((32, BC), jnp.float32),
            pltpu.VMEM((32, BC), jnp.float32),
            pltpu.SemaphoreType.DMA,
            pltpu.SemaphoreType.DMA,
            pltpu.SemaphoreType.DMA,
            pltpu.SemaphoreType.DMA,
        ],
    )(xT, z)
    return outT


def kernel(x, tables):
    tabT = jnp.transpose(tables, (0, 2, 1))
    xT = jnp.transpose(x, (1, 0))
    outT = _multi_embedding(xT, tabT)
    return jnp.transpose(outT, (1, 0))


# speedup vs baseline: 1.9755x; 1.7583x over previous
import functools

import jax
import jax.numpy as jnp
from jax import lax
from jax.experimental import pallas as pl
from jax.experimental.pallas import tpu as pltpu
from jax.experimental.pallas import tpu_sc as plsc

F = 26
V = 100000
D = 32
B = 16384
TOT = B * F
NC, NS, L = 2, 16, 16
NW = NC * NS
BPW = B // NW
PER_W = TOT // NW
SUP = 832
NSUP = PER_W // SUP


def _emb_body(xT_hbm, tab_hbm, out_hbm, xbufT, idxbuf, rb0, rb1,
              gs0, gs1, cs0, cs1):
    wid = lax.axis_index("s") * NC + lax.axis_index("c")
    base = wid * PER_W

    pltpu.sync_copy(xT_hbm.at[:, pl.ds(wid * BPW, BPW)], xbufT)

    def bidx(b, carry):
        for f0 in (0, 10):
            fv = f0 + lax.iota(jnp.int32, L)
            xv = plsc.load_gather(xbufT, [fv, jnp.full((L,), b, jnp.int32)])
            idxbuf[pl.ds(b * F + f0, L)] = xv + fv * V
        return carry

    lax.fori_loop(0, BPW, bidx, 0)

    def gather(s, rb, sem):
        return pltpu.make_async_copy(
            tab_hbm.at[idxbuf.at[pl.ds(s * SUP, SUP)]], rb, sem)

    def out_copy(s, rb, sem):
        return pltpu.make_async_copy(
            rb, out_hbm.at[pl.ds(base + s * SUP, SUP)], sem)

    gather(0, rb0, gs0).start()
    gather(1, rb1, gs1).start()

    for s in range(NSUP):
        rb, gs, cs = (rb0, gs0, cs0) if s % 2 == 0 else (rb1, gs1, cs1)
        gather(s, rb, gs).wait()
        out_copy(s, rb, cs).start()
        if s + 2 < NSUP:
            out_copy(s, rb, cs).wait()
            gather(s + 2, rb, gs).start()

    out_copy(NSUP - 2, rb0, cs0).wait()
    out_copy(NSUP - 1, rb1, cs1).wait()


@jax.jit
def _multi_embedding(xT, tab_flat):
    mesh = plsc.VectorSubcoreMesh(core_axis_name="c", subcore_axis_name="s")
    run = pl.kernel(
        _emb_body,
        mesh=mesh,
        compiler_params=pltpu.CompilerParams(use_tc_tiling_on_sc=False,
                                             needs_layout_passes=False),
        out_type=jax.ShapeDtypeStruct((TOT, D), jnp.float32),
        scratch_types=[
            pltpu.VMEM((F, BPW), jnp.int32),
            pltpu.VMEM((PER_W,), jnp.int32),
            pltpu.VMEM((SUP, D), jnp.float32),
            pltpu.VMEM((SUP, D), jnp.float32),
            pltpu.SemaphoreType.DMA,
            pltpu.SemaphoreType.DMA,
            pltpu.SemaphoreType.DMA,
            pltpu.SemaphoreType.DMA,
        ],
    )
    return run(xT, tab_flat)


def kernel(x, tables):
    xT = jnp.transpose(x, (1, 0))
    tab_flat = tables.reshape(F * V, D)
    out = _multi_embedding(xT, tab_flat)
    return out.reshape(B, F * D)


# speedup vs baseline: 1.9935x; 1.0091x over previous
import functools

import jax
import jax.numpy as jnp
from jax import lax
from jax.experimental import pallas as pl
from jax.experimental.pallas import tpu as pltpu
from jax.experimental.pallas import tpu_sc as plsc

F = 26
V = 100000
D = 32
B = 16384
TOT = B * F
NC, NS, L = 2, 16, 16
NW = NC * NS
PER_W = TOT // NW
IDXW = 128
ROWS_W = PER_W // IDXW
SUPR = 13
SUP = SUPR * IDXW
NSUP = ROWS_W // SUPR


def _emb_body(x_hbm, tab_hbm, out_hbm, idxbuf, rb0, rb1, gs0, gs1, cs0, cs1):
    wid = lax.axis_index("s") * NC + lax.axis_index("c")
    base = wid * PER_W

    pltpu.sync_copy(x_hbm.at[pl.ds(base, PER_W)], idxbuf)

    def compute_idx(s):
        def row(r, c):
            for v in range(IDXW // L):
                pos = base + r * IDXW + (v * L + lax.iota(jnp.int32, L))
                sl = pl.ds(r * IDXW + v * L, L)
                idxbuf[sl] = idxbuf[sl] + lax.rem(pos, F) * V
            return c
        lax.fori_loop(s * SUPR, (s + 1) * SUPR, row, 0)

    def gather(s, rb, sem):
        return pltpu.make_async_copy(
            tab_hbm.at[idxbuf.at[pl.ds(s * SUP, SUP)]], rb, sem)

    def out_copy(s, rb, sem):
        return pltpu.make_async_copy(
            rb, out_hbm.at[pl.ds(base + s * SUP, SUP)], sem)

    compute_idx(0)
    gather(0, rb0, gs0).start()
    compute_idx(1)
    gather(1, rb1, gs1).start()

    for s in range(NSUP):
        rb, gs, cs = (rb0, gs0, cs0) if s % 2 == 0 else (rb1, gs1, cs1)
        gather(s, rb, gs).wait()
        out_copy(s, rb, cs).start()
        if s + 2 < NSUP:
            compute_idx(s + 2)
            out_copy(s, rb, cs).wait()
            gather(s + 2, rb, gs).start()

    out_copy(NSUP - 2, rb0, cs0).wait()
    out_copy(NSUP - 1, rb1, cs1).wait()


@jax.jit
def _multi_embedding(x2d, tab_flat):
    mesh = plsc.VectorSubcoreMesh(core_axis_name="c", subcore_axis_name="s")
    run = functools.partial(
        pl.kernel,
        mesh=mesh,
        compiler_params=pltpu.CompilerParams(use_tc_tiling_on_sc=False),
        out_type=jax.ShapeDtypeStruct((TOT, D), jnp.float32),
        scratch_types=[
            pltpu.VMEM((PER_W,), jnp.int32),
            pltpu.VMEM((SUP, D), jnp.float32),
            pltpu.VMEM((SUP, D), jnp.float32),
            pltpu.SemaphoreType.DMA,
            pltpu.SemaphoreType.DMA,
            pltpu.SemaphoreType.DMA,
            pltpu.SemaphoreType.DMA,
        ],
    )(_emb_body)
    return run(x2d, tab_flat)


def kernel(x, tables):
    x2d = x.reshape(TOT)
    tab_flat = tables.reshape(F * V, D)
    out = _multi_embedding(x2d, tab_flat)
    return out.reshape(B, F * D)
